# baseline (speedup 1.0000x reference)
import functools

import jax
import jax.numpy as jnp
from jax import lax
from jax.experimental import pallas as pl
from jax.experimental.pallas import tpu as pltpu
from jax.experimental.pallas import tpu_sc as plsc

NUM_EMB = 10
EMB_DIM = 5
BATCH = 16384
HIST = 200

NC = 2
NS = 16
NW = NC * NS
BW = BATCH // NW
LCH = 8
NCHUNK = HIST // LCH
LANES = 16
NV = BW // LANES
UNROLL = 2


def _sc_body(idx_hbm, tab_hbm, out_hbm, tab_v,
             idx_v0, idx_v1, out_v0, out_v1,
             sem_i0, sem_i1, sem_o0, sem_o1):
    cid = lax.axis_index("c")
    sid = lax.axis_index("s")
    wid = sid * NC + cid
    b0 = wid * BW

    pltpu.sync_copy(tab_hbm, tab_v)
    for j in range(4):
        sl = pl.ds(j * LANES, LANES)
        tab_v[sl] = jnp.maximum(tab_v[sl], 0.0)

    in_bufs = (idx_v0, idx_v1)
    out_bufs = (out_v0, out_v1)
    sem_in = (sem_i0, sem_i1)
    sem_out = (sem_o0, sem_o1)

    def start_in(c):
        b = c % 2
        return pltpu.async_copy(
            idx_hbm.at[pl.ds(c * LCH, LCH), pl.ds(b0, BW)],
            in_bufs[b], sem_in[b])

    def start_out(c):
        b = c % 2
        return pltpu.async_copy(
            out_bufs[b],
            out_hbm.at[:, pl.ds(c * LCH, LCH), pl.ds(b0, BW)],
            sem_out[b])

    def compute(in_b, out_b):
        @plsc.parallel_loop(0, LCH * NV, unroll=UNROLL)
        def body(i):
            l = i >> 5
            v = (i & (NV - 1)) * LANES
            iv5 = in_b[l, pl.ds(v, LANES)] * 5
            for d in range(EMB_DIM):
                out_b[d, l, pl.ds(v, LANES)] = plsc.load_gather(
                    tab_v, [iv5 + d])

    in_copies = {0: start_in(0)}
    out_copies = {}
    for c in range(NCHUNK):
        b = c % 2
        in_copies[c].wait()
        if c + 1 < NCHUNK:
            in_copies[c + 1] = start_in(c + 1)
        if c >= 2:
            out_copies[c - 2].wait()
        compute(in_bufs[b], out_bufs[b])
        out_copies[c] = start_out(c)
    out_copies[NCHUNK - 2].wait()
    out_copies[NCHUNK - 1].wait()


@jax.jit
def _lookup(idx_t, tab_flat):
    mesh = plsc.VectorSubcoreMesh(core_axis_name="c", subcore_axis_name="s")
    f = functools.partial(
        pl.kernel,
        mesh=mesh,
        out_type=jax.ShapeDtypeStruct((EMB_DIM, HIST, BATCH), jnp.float32),
        scratch_types=[
            pltpu.VMEM((64,), jnp.float32),
            pltpu.VMEM((LCH, BW), jnp.int32),
            pltpu.VMEM((LCH, BW), jnp.int32),
            pltpu.VMEM((EMB_DIM, LCH, BW), jnp.float32),
            pltpu.VMEM((EMB_DIM, LCH, BW), jnp.float32),
            pltpu.SemaphoreType.DMA,
            pltpu.SemaphoreType.DMA,
            pltpu.SemaphoreType.DMA,
            pltpu.SemaphoreType.DMA,
        ],
        compiler_params=pltpu.CompilerParams(needs_layout_passes=False),
    )(_sc_body)
    return f(idx_t, tab_flat)


def kernel(indices, table):
    tab_flat = jnp.pad(table.reshape(-1), (0, 64 - NUM_EMB * EMB_DIM))
    idx_t = jnp.transpose(indices.astype(jnp.int32))
    out_t = _lookup(idx_t, tab_flat)
    return jnp.transpose(out_t, (2, 1, 0))


# speedup vs baseline: 1.0048x; 1.0048x over previous
import functools

import jax
import jax.numpy as jnp
from jax import lax
from jax.experimental import pallas as pl
from jax.experimental.pallas import tpu as pltpu
from jax.experimental.pallas import tpu_sc as plsc

NUM_EMB = 10
EMB_DIM = 5
BATCH = 16384
HIST = 200

NC = 2
NS = 16
NW = NC * NS
BW = BATCH // NW
LCH = 8
NCHUNK = HIST // LCH
LANES = 16
NV = BW // LANES
UNROLL = 4


def _sc_body(idx_hbm, tab_hbm, out_hbm, tab_v,
             idx_v0, idx_v1, out_v0, out_v1,
             sem_i0, sem_i1, sem_o0, sem_o1):
    cid = lax.axis_index("c")
    sid = lax.axis_index("s")
    wid = sid * NC + cid
    b0 = wid * BW

    pltpu.sync_copy(tab_hbm, tab_v)
    for j in range(4):
        sl = pl.ds(j * LANES, LANES)
        tab_v[sl] = jnp.maximum(tab_v[sl], 0.0)

    in_bufs = (idx_v0, idx_v1)
    out_bufs = (out_v0, out_v1)
    sem_in = (sem_i0, sem_i1)
    sem_out = (sem_o0, sem_o1)

    def start_in(c):
        b = c % 2
        return pltpu.async_copy(
            idx_hbm.at[pl.ds(c * LCH, LCH), pl.ds(b0, BW)],
            in_bufs[b], sem_in[b])

    def start_out(c):
        b = c % 2
        return pltpu.async_copy(
            out_bufs[b],
            out_hbm.at[:, pl.ds(c * LCH, LCH), pl.ds(b0, BW)],
            sem_out[b])

    def compute(in_b, out_b):
        @plsc.parallel_loop(0, LCH * NV, unroll=UNROLL)
        def body(i):
            l = i >> 5
            v = (i & (NV - 1)) * LANES
            iv5 = in_b[l, pl.ds(v, LANES)] * 5
            for d in range(EMB_DIM):
                out_b[d, l, pl.ds(v, LANES)] = plsc.load_gather(
                    tab_v, [iv5 + d])

    in_copies = {0: start_in(0)}
    out_copies = {}
    for c in range(NCHUNK):
        b = c % 2
        in_copies[c].wait()
        if c + 1 < NCHUNK:
            in_copies[c + 1] = start_in(c + 1)
        if c >= 2:
            out_copies[c - 2].wait()
        compute(in_bufs[b], out_bufs[b])
        out_copies[c] = start_out(c)
    out_copies[NCHUNK - 2].wait()
    out_copies[NCHUNK - 1].wait()


@jax.jit
def _lookup(idx_t, tab_flat):
    mesh = plsc.VectorSubcoreMesh(core_axis_name="c", subcore_axis_name="s")
    f = functools.partial(
        pl.kernel,
        mesh=mesh,
        out_type=jax.ShapeDtypeStruct((EMB_DIM, HIST, BATCH), jnp.float32),
        scratch_types=[
            pltpu.VMEM((64,), jnp.float32),
            pltpu.VMEM((LCH, BW), jnp.int32),
            pltpu.VMEM((LCH, BW), jnp.int32),
            pltpu.VMEM((EMB_DIM, LCH, BW), jnp.float32),
            pltpu.VMEM((EMB_DIM, LCH, BW), jnp.float32),
            pltpu.SemaphoreType.DMA,
            pltpu.SemaphoreType.DMA,
            pltpu.SemaphoreType.DMA,
            pltpu.SemaphoreType.DMA,
        ],
        compiler_params=pltpu.CompilerParams(needs_layout_passes=False),
    )(_sc_body)
    return f(idx_t, tab_flat)


def kernel(indices, table):
    tab_flat = jnp.pad(table.reshape(-1), (0, 64 - NUM_EMB * EMB_DIM))
    idx_t = jnp.transpose(indices.astype(jnp.int32))
    out_t = _lookup(idx_t, tab_flat)
    return jnp.transpose(out_t, (2, 1, 0))


# speedup vs baseline: 1.0089x; 1.0041x over previous
import functools

import jax
import jax.numpy as jnp
from jax import lax
from jax.experimental import pallas as pl
from jax.experimental.pallas import tpu as pltpu
from jax.experimental.pallas import tpu_sc as plsc

NUM_EMB = 10
EMB_DIM = 5
BATCH = 16384
HIST = 200

NC = 2
NS = 16
NW = NC * NS
BW = BATCH // NW
LCH = 8
NCHUNK = HIST // LCH
LANES = 16
NV = BW // LANES
UNROLL = 4


def _sc_body(idx_hbm, tab_hbm, out_hbm, tab_v,
             idx_v0, idx_v1, out_v0, out_v1,
             sem_i0, sem_i1, sem_o0, sem_o1):
    cid = lax.axis_index("c")
    sid = lax.axis_index("s")
    wid = sid * NC + cid
    b0 = wid * BW

    pltpu.sync_copy(tab_hbm, tab_v)
    for j in range(4):
        sl = pl.ds(j * LANES, LANES)
        tab_v[sl] = jnp.maximum(tab_v[sl], 0.0)

    in_bufs = (idx_v0, idx_v1)
    out_bufs = (out_v0, out_v1)
    sem_in = (sem_i0, sem_i1)
    sem_out = (sem_o0, sem_o1)

    def start_in(c):
        b = c % 2
        return pltpu.async_copy(
            idx_hbm.at[pl.ds(c * LCH, LCH), pl.ds(b0, BW)],
            in_bufs[b], sem_in[b])

    def start_out(c):
        b = c % 2
        return pltpu.async_copy(
            out_bufs[b],
            out_hbm.at[:, pl.ds(c * LCH, LCH), pl.ds(b0, BW)],
            sem_out[b])

    def compute(in_b, out_b):
        @plsc.parallel_loop(0, LCH * NV, unroll=UNROLL)
        def body(i):
            l = i >> 5
            v = (i & (NV - 1)) * LANES
            iv5 = in_b[l, pl.ds(v, LANES)] * 5
            for d in range(EMB_DIM):
                out_b[d, l, pl.ds(v, LANES)] = plsc.load_gather(
                    tab_v, [iv5 + d])

    in_copies = {0: start_in(0)}
    out_copies = {}
    for c in range(NCHUNK):
        b = c % 2
        in_copies[c].wait()
        if c + 1 < NCHUNK:
            in_copies[c + 1] = start_in(c + 1)
        if c >= 2:
            out_copies[c - 2].wait()
        compute(in_bufs[b], out_bufs[b])
        out_copies[c] = start_out(c)
    out_copies[NCHUNK - 2].wait()
    out_copies[NCHUNK - 1].wait()


@jax.jit
def _lookup(idx_t, tab_flat):
    mesh = plsc.VectorSubcoreMesh(core_axis_name="c", subcore_axis_name="s")
    f = functools.partial(
        pl.kernel,
        mesh=mesh,
        out_type=jax.ShapeDtypeStruct((EMB_DIM, HIST, BATCH), jnp.float32),
        scratch_types=[
            pltpu.VMEM((64,), jnp.float32),
            pltpu.VMEM((LCH, BW), jnp.int32),
            pltpu.VMEM((LCH, BW), jnp.int32),
            pltpu.VMEM((EMB_DIM, LCH, BW), jnp.float32),
            pltpu.VMEM((EMB_DIM, LCH, BW), jnp.float32),
            pltpu.SemaphoreType.DMA,
            pltpu.SemaphoreType.DMA,
            pltpu.SemaphoreType.DMA,
            pltpu.SemaphoreType.DMA,
        ],
        compiler_params=pltpu.CompilerParams(needs_layout_passes=False, disable_bounds_checks=True, disable_semaphore_checks=True),
    )(_sc_body)
    return f(idx_t, tab_flat)


def kernel(indices, table):
    tab_flat = jnp.pad(table.reshape(-1), (0, 64 - NUM_EMB * EMB_DIM))
    idx_t = jnp.transpose(indices.astype(jnp.int32))
    out_t = _lookup(idx_t, tab_flat)
    return jnp.transpose(out_t, (2, 1, 0))


# speedup vs baseline: 1.1636x; 1.1533x over previous
import functools

import jax
import jax.numpy as jnp
from jax import lax
from jax.experimental import pallas as pl
from jax.experimental.pallas import tpu as pltpu
from jax.experimental.pallas import tpu_sc as plsc

NUM_EMB = 10
EMB_DIM = 5
BATCH = 16384
HIST = 200

NC = 2
NS = 16
NW = NC * NS
BW = BATCH // NW
LCH = 8
NCHUNK = HIST // LCH
LANES = 16
NV = BW // LANES
UNROLL = 4


def _sc_body(idx_hbm, tab_hbm, out_hbm, tab_v,
             idx_v0, idx_v1, out_v0, out_v1,
             sem_i0, sem_i1, sem_o0, sem_o1):
    cid = lax.axis_index("c")
    sid = lax.axis_index("s")
    wid = sid * NC + cid
    b0 = wid * BW

    pltpu.sync_copy(tab_hbm, tab_v)
    for j in range(4):
        sl = pl.ds(j * LANES, LANES)
        tab_v[sl] = jnp.maximum(tab_v[sl], 0.0)

    in_bufs = (idx_v0, idx_v1)
    out_bufs = (out_v0, out_v1)
    sem_in = (sem_i0, sem_i1)
    sem_out = (sem_o0, sem_o1)

    def start_in(c, b):
        return pltpu.async_copy(
            idx_hbm.at[pl.ds(c * LCH, LCH), pl.ds(b0, BW)],
            in_bufs[b], sem_in[b])

    def start_out(c, b):
        return pltpu.async_copy(
            out_bufs[b],
            out_hbm.at[:, pl.ds(c * LCH, LCH), pl.ds(b0, BW)],
            sem_out[b])

    def compute(in_b, out_b):
        @plsc.parallel_loop(0, LCH * NV, unroll=UNROLL)
        def body(i):
            l = i >> 5
            v = (i & (NV - 1)) * LANES
            iv5 = in_b[l, pl.ds(v, LANES)] * 5
            for d in range(EMB_DIM):
                out_b[d, l, pl.ds(v, LANES)] = plsc.load_gather(
                    tab_v, [iv5 + d])

    def wait_in(c, b):
        pltpu.make_async_copy(
            idx_hbm.at[pl.ds(c * LCH, LCH), pl.ds(b0, BW)],
            in_bufs[b], sem_in[b]).wait()

    def wait_out(c, b):
        pltpu.make_async_copy(
            out_bufs[b],
            out_hbm.at[:, pl.ds(c * LCH, LCH), pl.ds(b0, BW)],
            sem_out[b]).wait()

    start_in(0, 0)
    start_in(1, 1)

    @pl.loop(0, NCHUNK - 1, step=2)
    def _chunks(g):
        for sub in range(2):
            c = g + sub
            wait_in(c, sub)

            @pl.when(c >= 2)
            def _drain():
                wait_out(c - 2, sub)

            compute(in_bufs[sub], out_bufs[sub])
            start_out(c, sub)

            @pl.when(c + 2 < NCHUNK)
            def _prefetch():
                start_in(c + 2, sub)

    c = NCHUNK - 1
    wait_in(c, 0)
    wait_out(c - 2, 0)
    compute(in_bufs[0], out_bufs[0])
    start_out(c, 0)
    wait_out(NCHUNK - 2, 1)
    wait_out(NCHUNK - 1, 0)


@jax.jit
def _lookup(idx_t, tab_flat):
    mesh = plsc.VectorSubcoreMesh(core_axis_name="c", subcore_axis_name="s")
    f = functools.partial(
        pl.kernel,
        mesh=mesh,
        out_type=jax.ShapeDtypeStruct((EMB_DIM, HIST, BATCH), jnp.float32),
        scratch_types=[
            pltpu.VMEM((64,), jnp.float32),
            pltpu.VMEM((LCH, BW), jnp.int32),
            pltpu.VMEM((LCH, BW), jnp.int32),
            pltpu.VMEM((EMB_DIM, LCH, BW), jnp.float32),
            pltpu.VMEM((EMB_DIM, LCH, BW), jnp.float32),
            pltpu.SemaphoreType.DMA,
            pltpu.SemaphoreType.DMA,
            pltpu.SemaphoreType.DMA,
            pltpu.SemaphoreType.DMA,
        ],
        compiler_params=pltpu.CompilerParams(needs_layout_passes=False, disable_bounds_checks=True, disable_semaphore_checks=True),
    )(_sc_body)
    return f(idx_t, tab_flat)


def kernel(indices, table):
    tab_flat = jnp.pad(table.reshape(-1), (0, 64 - NUM_EMB * EMB_DIM))
    idx_t = jnp.transpose(indices.astype(jnp.int32))
    out_t = _lookup(idx_t, tab_flat)
    return jnp.transpose(out_t, (2, 1, 0))
